# baseline (speedup 1.0000x reference)
import functools

import jax
import jax.numpy as jnp
from jax import lax
from jax.experimental import pallas as pl
from jax.experimental.pallas import tpu as pltpu
from jax.experimental.pallas import tpu_sc as plsc

B, C, H, W = 4, 19, 512, 512
P = B * H * W
NB = 1024
NW = 32
HSIZE = 2 * C * NB
RB = 128
NRB = H // RB
NBP = 1
NPARTS = B // NBP
ROWSP = C * NBP * H
W2 = W // 2
WROWS = ROWSP // NW
BR = 8
NBLK = WROWS // BR


def _prep_body(x_ref, t_ref, ow_ref, ex_ref, inv_ref):
    c = pl.program_id(2)

    @pl.when(c == 0)
    def _():
        x = x_ref[0]
        m = jnp.max(x, axis=0)
        ex = jnp.exp(x - m[None])
        ex_ref[...] = ex
        inv_ref[...] = 1.0 / jnp.sum(ex, axis=0)

    p = ex_ref[c] * inv_ref[...]
    fg = t_ref[0] == c
    e = jnp.where(fg, 1.0 - p, p)
    bin_ = jnp.minimum((e * NB).astype(jnp.int32), NB - 1)
    idx = jnp.where(fg, C * NB, 0) + c * NB + bin_
    ow_ref[...] = idx[:, :W2] | lax.shift_left(idx[:, W2:], 16)


def _prep(x, t, b0):
    return pl.pallas_call(
        _prep_body,
        grid=(NBP, NRB, C),
        in_specs=[
            pl.BlockSpec((1, C, RB, W), lambda b, r, c: (b0 + b, 0, r, 0)),
            pl.BlockSpec((1, RB, W), lambda b, r, c: (b0 + b, r, 0)),
        ],
        out_specs=pl.BlockSpec((RB, W2), lambda b, r, c: ((b * NRB + r) * C + c, 0)),
        out_shape=jax.ShapeDtypeStruct((ROWSP, W2), jnp.int32),
        scratch_shapes=[
            pltpu.VMEM((C, RB, W), jnp.float32),
            pltpu.VMEM((RB, W), jnp.float32),
        ],
    )(x, t)


def _sc_hist_body(w_hbm, out_hbm, hist, bufw, sems):
    wid = lax.axis_index("s") * 2 + lax.axis_index("c")
    row0 = wid * WROWS

    zero = jnp.zeros((16,), jnp.float32)

    def zbody(i, carry):
        hist[pl.ds(i * 64, 16)] = zero
        hist[pl.ds(i * 64 + 16, 16)] = zero
        hist[pl.ds(i * 64 + 32, 16)] = zero
        hist[pl.ds(i * 64 + 48, 16)] = zero
        return carry

    lax.fori_loop(0, HSIZE // 64, zbody, 0)

    ones = jnp.ones((16,), jnp.float32)
    lomask = jnp.int32(0xFFFF)

    def start(blk, buf):
        r = row0 + blk * BR
        pltpu.async_copy(w_hbm.at[pl.ds(r, BR), :], bufw.at[buf], sems.at[buf])

    def wait(blk, buf):
        r = row0 + blk * BR
        pltpu.make_async_copy(w_hbm.at[pl.ds(r, BR), :], bufw.at[buf],
                              sems.at[buf]).wait()

    start(0, 0)

    def block_loop(q, carry):
        for par in range(2):
            blk = q * 2 + par

            @pl.when(blk + 1 < NBLK)
            def _():
                start(blk + 1, 1 - par)

            wait(blk, par)

            def row_loop(r, carry2, par=par):
                @plsc.parallel_loop(0, W2 // 16, 1, unroll=8)
                def vec_loop(j, r=r, par=par):
                    wv = bufw[par, r, pl.ds(j * 16, 16)]
                    i1 = wv & lomask
                    i2 = lax.shift_right_logical(wv, 16)
                    plsc.addupdate_scatter(hist, [i1], ones)
                    plsc.addupdate_scatter(hist, [i2], ones)

                return carry2

            lax.fori_loop(0, BR, row_loop, 0)
        return carry

    lax.fori_loop(0, NBLK // 2, block_loop, 0)

    pltpu.sync_copy(hist, out_hbm.at[wid])


def _sc_hist(w2):
    mesh = plsc.VectorSubcoreMesh(core_axis_name="c", subcore_axis_name="s")
    kern = functools.partial(
        pl.kernel,
        out_type=jax.ShapeDtypeStruct((NW, HSIZE), jnp.float32),
        mesh=mesh,
        compiler_params=pltpu.CompilerParams(needs_layout_passes=False),
        scratch_types=[
            pltpu.VMEM((HSIZE,), jnp.float32),
            pltpu.VMEM((2, BR, W2), jnp.int32),
            pltpu.SemaphoreType.DMA((2,)),
        ],
    )(_sc_hist_body)
    return kern(w2)


def _finalize_body(h_ref, o_ref):
    h = h_ref[...]
    s = jnp.sum(h, axis=0)
    n_bg, n_fg = s[0], s[1]

    mid = (lax.broadcasted_iota(jnp.int32, (C, NB), 1).astype(jnp.float32)
           + 0.5) * (1.0 / NB)
    s_bg = n_bg * mid
    s_fg = n_fg * mid

    r = lax.broadcasted_iota(jnp.int32, (NB, NB), 0)
    col = lax.broadcasted_iota(jnp.int32, (NB, NB), 1)
    upper = (r > col).astype(jnp.float32)

    n_all = n_fg + n_bg
    K = jnp.dot(n_all, upper, preferred_element_type=jnp.float32)
    CF = jnp.dot(n_fg, upper, preferred_element_type=jnp.float32)
    G = jnp.sum(n_fg, axis=1, keepdims=True)

    A = jnp.maximum(G + K - CF, 0.5)
    contrib = s_fg / A + s_bg * (G - CF - n_fg) / (A * (A + n_bg))
    loss = jnp.sum(contrib, axis=1, keepdims=True)

    edge = (lax.broadcasted_iota(jnp.int32, (C, NB), 1).astype(jnp.float32)
            + 1.0) * (1.0 / NB)
    emax = jnp.max(jnp.where(n_all > 0, edge, 0.0), axis=1, keepdims=True)
    loss = jnp.where(G > 0, loss, emax)

    o_ref[...] = jnp.sum(loss, axis=(0, 1), keepdims=True) * (1.0 / C)


def _finalize(h4):
    return pl.pallas_call(
        _finalize_body,
        out_shape=jax.ShapeDtypeStruct((1, 1), jnp.float32),
    )(h4)


def kernel(input, target):
    t = target.astype(jnp.int32)
    hs = []
    for part in range(NPARTS):
        w_p = _prep(input, t, part * NBP)
        hs.append(_sc_hist(w_p))
    h4 = jnp.concatenate(hs, axis=0).reshape(NPARTS * NW, 2, C, NB)
    out = _finalize(h4)
    return out.reshape(())


# speedup vs baseline: 1.1487x; 1.1487x over previous
import functools

import jax
import jax.numpy as jnp
from jax import lax
from jax.experimental import pallas as pl
from jax.experimental.pallas import tpu as pltpu
from jax.experimental.pallas import tpu_sc as plsc

B, C, H, W = 4, 19, 512, 512
P = B * H * W
NB = 1024
NW = 32
HSIZE = 2 * C * NB
RB = 128
NRB = H // RB
NBP = 2
NPARTS = B // NBP
ROWSP = C * NBP * H
RALL = NBP * NRB
W2 = W // 2
WROWS = ROWSP // NW
BR = 16
NBLK = WROWS // BR


def _softmax_stash(x, ex_ref, inv_ref):
    m = jnp.max(x, axis=0)
    ex = jnp.exp(x - m[None])
    ex_ref[...] = ex
    inv_ref[...] = 1.0 / jnp.sum(ex, axis=0)


def _prep_body(xa_ref, xb_ref, t_ref, ow_ref, ex_ref, inv_ref):
    q = pl.program_id(0)
    c = pl.program_id(1)
    even = (q & 1) == 0

    @pl.when((c == 0) & even)
    def _():
        _softmax_stash(xa_ref[0], ex_ref, inv_ref)

    @pl.when((c == 0) & jnp.logical_not(even))
    def _():
        _softmax_stash(xb_ref[0], ex_ref, inv_ref)

    p = ex_ref[c] * inv_ref[...]
    fg = t_ref[0] == c
    e = jnp.where(fg, 1.0 - p, p)
    bin_ = jnp.minimum((e * NB).astype(jnp.int32), NB - 1)
    idx = jnp.where(fg, C * NB, 0) + c * NB + bin_
    ow_ref[...] = idx[:, :W2] | lax.shift_left(idx[:, W2:], 16)


def _prep(x, t, b0):
    def _amap(q, c):
        qa = jnp.minimum((q + 1) // 2 * 2, RALL - 2)
        return (b0 + qa // NRB, 0, qa % NRB, 0)

    def _bmap(q, c):
        qb = jnp.minimum(q // 2 * 2 + 1, RALL - 1)
        return (b0 + qb // NRB, 0, qb % NRB, 0)

    return pl.pallas_call(
        _prep_body,
        grid=(RALL, C),
        in_specs=[
            pl.BlockSpec((1, C, RB, W), _amap),
            pl.BlockSpec((1, C, RB, W), _bmap),
            pl.BlockSpec((1, RB, W), lambda q, c: (b0 + q // NRB, q % NRB, 0)),
        ],
        out_specs=pl.BlockSpec((RB, W2), lambda q, c: (q * C + c, 0)),
        out_shape=jax.ShapeDtypeStruct((ROWSP, W2), jnp.int32),
        scratch_shapes=[
            pltpu.VMEM((C, RB, W), jnp.float32),
            pltpu.VMEM((RB, W), jnp.float32),
        ],
    )(x, x, t)


def _sc_hist_body(w_hbm, out_hbm, hist, bufw, sems):
    wid = lax.axis_index("s") * 2 + lax.axis_index("c")
    row0 = wid * WROWS

    zero = jnp.zeros((16,), jnp.float32)

    def zbody(i, carry):
        hist[pl.ds(i * 64, 16)] = zero
        hist[pl.ds(i * 64 + 16, 16)] = zero
        hist[pl.ds(i * 64 + 32, 16)] = zero
        hist[pl.ds(i * 64 + 48, 16)] = zero
        return carry

    lax.fori_loop(0, HSIZE // 64, zbody, 0)

    ones = jnp.ones((16,), jnp.float32)
    lomask = jnp.int32(0xFFFF)

    def start(blk, buf):
        r = row0 + blk * BR
        pltpu.async_copy(w_hbm.at[pl.ds(r, BR), :], bufw.at[buf], sems.at[buf])

    def wait(blk, buf):
        r = row0 + blk * BR
        pltpu.make_async_copy(w_hbm.at[pl.ds(r, BR), :], bufw.at[buf],
                              sems.at[buf]).wait()

    start(0, 0)

    def block_loop(q, carry):
        for par in range(2):
            blk = q * 2 + par

            @pl.when(blk + 1 < NBLK)
            def _():
                start(blk + 1, 1 - par)

            wait(blk, par)

            def row_loop(r, carry2, par=par):
                @plsc.parallel_loop(0, W2 // 16, 1, unroll=8)
                def vec_loop(j, r=r, par=par):
                    wv = bufw[par, r, pl.ds(j * 16, 16)]
                    i1 = wv & lomask
                    i2 = lax.shift_right_logical(wv, 16)
                    plsc.addupdate_scatter(hist, [i1], ones)
                    plsc.addupdate_scatter(hist, [i2], ones)

                return carry2

            lax.fori_loop(0, BR, row_loop, 0)
        return carry

    lax.fori_loop(0, NBLK // 2, block_loop, 0)

    pltpu.sync_copy(hist, out_hbm.at[wid])


def _sc_hist(w2):
    mesh = plsc.VectorSubcoreMesh(core_axis_name="c", subcore_axis_name="s")
    kern = functools.partial(
        pl.kernel,
        out_type=jax.ShapeDtypeStruct((NW, HSIZE), jnp.float32),
        mesh=mesh,
        compiler_params=pltpu.CompilerParams(needs_layout_passes=False),
        scratch_types=[
            pltpu.VMEM((HSIZE,), jnp.float32),
            pltpu.VMEM((2, BR, W2), jnp.int32),
            pltpu.SemaphoreType.DMA((2,)),
        ],
    )(_sc_hist_body)
    return kern(w2)


def _finalize_body(ha_ref, hb_ref, o_ref):
    s = jnp.sum(ha_ref[...], axis=0) + jnp.sum(hb_ref[...], axis=0)
    n_bg, n_fg = s[0], s[1]

    mid = (lax.broadcasted_iota(jnp.int32, (C, NB), 1).astype(jnp.float32)
           + 0.5) * (1.0 / NB)
    s_bg = n_bg * mid
    s_fg = n_fg * mid

    r = lax.broadcasted_iota(jnp.int32, (NB, NB), 0)
    col = lax.broadcasted_iota(jnp.int32, (NB, NB), 1)
    upper = (r > col).astype(jnp.float32)

    n_all = n_fg + n_bg
    K = jnp.dot(n_all, upper, preferred_element_type=jnp.float32)
    CF = jnp.dot(n_fg, upper, preferred_element_type=jnp.float32)
    G = jnp.sum(n_fg, axis=1, keepdims=True)

    A = jnp.maximum(G + K - CF, 0.5)
    contrib = s_fg / A + s_bg * (G - CF - n_fg) / (A * (A + n_bg))
    loss = jnp.sum(contrib, axis=1, keepdims=True)

    edge = (lax.broadcasted_iota(jnp.int32, (C, NB), 1).astype(jnp.float32)
            + 1.0) * (1.0 / NB)
    emax = jnp.max(jnp.where(n_all > 0, edge, 0.0), axis=1, keepdims=True)
    loss = jnp.where(G > 0, loss, emax)

    o_ref[...] = jnp.sum(loss, axis=(0, 1), keepdims=True) * (1.0 / C)


def _finalize(h_a, h_b):
    return pl.pallas_call(
        _finalize_body,
        out_shape=jax.ShapeDtypeStruct((1, 1), jnp.float32),
    )(h_a, h_b)


def kernel(input, target):
    t = target.astype(jnp.int32)
    w_a = _prep(input, t, 0)
    h_a = _sc_hist(w_a)
    w_b = _prep(input, t, NBP)
    h_b = _sc_hist(w_b)
    out = _finalize(h_a.reshape(NW, 2, C, NB), h_b.reshape(NW, 2, C, NB))
    return out.reshape(())


# speedup vs baseline: 1.8233x; 1.5873x over previous
import functools

import jax
import jax.numpy as jnp
from jax import lax
from jax.experimental import pallas as pl
from jax.experimental.pallas import tpu as pltpu
from jax.experimental.pallas import tpu_sc as plsc

B, C, H, W = 4, 19, 512, 512
P = B * H * W
NB = 1024
NW = 32
HSIZE = 2 * C * NB
RB = 128
NRB = H // RB
NBP = 2
NPARTS = B // NBP
ROWSP = C * NBP * H
RALL = NBP * NRB
W2 = W // 2
WROWS = ROWSP // NW
BR = 16
NBLK = WROWS // BR


def _prep_body(x_ref, t_ref, ow_ref):
    x = x_ref[0]
    t = t_ref[0]
    m = jnp.max(x, axis=0)
    ex = jnp.exp(x - m[None])
    inv = 1.0 / jnp.sum(ex, axis=0)
    for c in range(C):
        p = ex[c] * inv
        fg = t == c
        e = jnp.where(fg, 1.0 - p, p)
        bin_ = jnp.minimum((e * NB).astype(jnp.int32), NB - 1)
        idx = jnp.where(fg, C * NB, 0) + c * NB + bin_
        ow_ref[pl.ds(c * RB, RB), :] = idx[:, :W2] | lax.shift_left(idx[:, W2:], 16)


def _prep(x, t, b0):
    return pl.pallas_call(
        _prep_body,
        grid=(RALL,),
        in_specs=[
            pl.BlockSpec((1, C, RB, W), lambda q: (b0 + q // NRB, 0, q % NRB, 0)),
            pl.BlockSpec((1, RB, W), lambda q: (b0 + q // NRB, q % NRB, 0)),
        ],
        out_specs=pl.BlockSpec((C * RB, W2), lambda q: (q, 0)),
        out_shape=jax.ShapeDtypeStruct((ROWSP, W2), jnp.int32),
    )(x, t)


def _sc_hist_body(w_hbm, out_hbm, hist, bufw, sems):
    wid = lax.axis_index("s") * 2 + lax.axis_index("c")
    row0 = wid * WROWS

    zero = jnp.zeros((16,), jnp.float32)

    def zbody(i, carry):
        hist[pl.ds(i * 64, 16)] = zero
        hist[pl.ds(i * 64 + 16, 16)] = zero
        hist[pl.ds(i * 64 + 32, 16)] = zero
        hist[pl.ds(i * 64 + 48, 16)] = zero
        return carry

    lax.fori_loop(0, HSIZE // 64, zbody, 0)

    ones = jnp.ones((16,), jnp.float32)
    lomask = jnp.int32(0xFFFF)

    def start(blk, buf):
        r = row0 + blk * BR
        pltpu.async_copy(w_hbm.at[pl.ds(r, BR), :], bufw.at[buf], sems.at[buf])

    def wait(blk, buf):
        r = row0 + blk * BR
        pltpu.make_async_copy(w_hbm.at[pl.ds(r, BR), :], bufw.at[buf],
                              sems.at[buf]).wait()

    start(0, 0)

    def block_loop(q, carry):
        for par in range(2):
            blk = q * 2 + par

            @pl.when(blk + 1 < NBLK)
            def _():
                start(blk + 1, 1 - par)

            wait(blk, par)

            def row_loop(r, carry2, par=par):
                @plsc.parallel_loop(0, W2 // 16, 1, unroll=8)
                def vec_loop(j, r=r, par=par):
                    wv = bufw[par, r, pl.ds(j * 16, 16)]
                    i1 = wv & lomask
                    i2 = lax.shift_right_logical(wv, 16)
                    plsc.addupdate_scatter(hist, [i1], ones)
                    plsc.addupdate_scatter(hist, [i2], ones)

                return carry2

            lax.fori_loop(0, BR, row_loop, 0)
        return carry

    lax.fori_loop(0, NBLK // 2, block_loop, 0)

    pltpu.sync_copy(hist, out_hbm.at[wid])


def _sc_hist(w2):
    mesh = plsc.VectorSubcoreMesh(core_axis_name="c", subcore_axis_name="s")
    kern = functools.partial(
        pl.kernel,
        out_type=jax.ShapeDtypeStruct((NW, HSIZE), jnp.float32),
        mesh=mesh,
        compiler_params=pltpu.CompilerParams(needs_layout_passes=False),
        scratch_types=[
            pltpu.VMEM((HSIZE,), jnp.float32),
            pltpu.VMEM((2, BR, W2), jnp.int32),
            pltpu.SemaphoreType.DMA((2,)),
        ],
    )(_sc_hist_body)
    return kern(w2)


def _finalize_body(ha_ref, hb_ref, o_ref):
    s = jnp.sum(ha_ref[...], axis=0) + jnp.sum(hb_ref[...], axis=0)
    n_bg, n_fg = s[0], s[1]

    mid = (lax.broadcasted_iota(jnp.int32, (C, NB), 1).astype(jnp.float32)
           + 0.5) * (1.0 / NB)
    s_bg = n_bg * mid
    s_fg = n_fg * mid

    r = lax.broadcasted_iota(jnp.int32, (NB, NB), 0)
    col = lax.broadcasted_iota(jnp.int32, (NB, NB), 1)
    upper = (r > col).astype(jnp.float32)

    n_all = n_fg + n_bg
    K = jnp.dot(n_all, upper, preferred_element_type=jnp.float32)
    CF = jnp.dot(n_fg, upper, preferred_element_type=jnp.float32)
    G = jnp.sum(n_fg, axis=1, keepdims=True)

    A = jnp.maximum(G + K - CF, 0.5)
    contrib = s_fg / A + s_bg * (G - CF - n_fg) / (A * (A + n_bg))
    loss = jnp.sum(contrib, axis=1, keepdims=True)

    edge = (lax.broadcasted_iota(jnp.int32, (C, NB), 1).astype(jnp.float32)
            + 1.0) * (1.0 / NB)
    emax = jnp.max(jnp.where(n_all > 0, edge, 0.0), axis=1, keepdims=True)
    loss = jnp.where(G > 0, loss, emax)

    o_ref[...] = jnp.sum(loss, axis=(0, 1), keepdims=True) * (1.0 / C)


def _finalize(h_a, h_b):
    return pl.pallas_call(
        _finalize_body,
        out_shape=jax.ShapeDtypeStruct((1, 1), jnp.float32),
    )(h_a, h_b)


def kernel(input, target):
    t = target.astype(jnp.int32)
    w_a = _prep(input, t, 0)
    h_a = _sc_hist(w_a)
    w_b = _prep(input, t, NBP)
    h_b = _sc_hist(w_b)
    out = _finalize(h_a.reshape(NW, 2, C, NB), h_b.reshape(NW, 2, C, NB))
    return out.reshape(())
